# baseline (speedup 1.0000x reference)
import functools

import jax
import jax.numpy as jnp
from jax import lax
from jax.experimental import pallas as pl
from jax.experimental.pallas import tpu as pltpu
from jax.experimental.pallas import tpu_sc as plsc

DIM = 256
KNB = 16


def _knn_pallas(pos, posT, interpret=False):
    B, N, _ = pos.shape
    TQ = 256

    def body(pos_ref, posT_ref, idx_ref):
        b = pl.program_id(0)
        pt = pos_ref[0]
        pT = posT_ref[0]
        dot = jnp.dot(pt, pT, preferred_element_type=jnp.float32)
        sq_r = jnp.sum(pT * pT, axis=0, keepdims=True)
        sq_t = jnp.sum(pt * pt, axis=1, keepdims=True)
        d2 = sq_t + sq_r - 2.0 * dot
        iota = lax.broadcasted_iota(jnp.int32, (TQ, N), 1).astype(jnp.float32)
        fn = jnp.float32(N)
        cols = []
        for _ in range(KNB):
            m = jnp.min(d2, axis=1, keepdims=True)
            cand = jnp.where(d2 == m, iota, fn)
            amin = jnp.min(cand, axis=1, keepdims=True)
            cols.append(amin)
            d2 = jnp.where(cand == amin, jnp.inf, d2)
        idxf = jnp.concatenate(cols, axis=1)
        idx_ref[0] = idxf.astype(jnp.int32) + b * N

    return pl.pallas_call(
        body,
        grid=(B, N // TQ),
        in_specs=[pl.BlockSpec((1, TQ, 3), lambda b, i: (b, i, 0)),
                  pl.BlockSpec((1, 3, N), lambda b, i: (b, 0, 0))],
        out_specs=pl.BlockSpec((1, TQ, KNB), lambda b, i: (b, i, 0)),
        out_shape=jax.ShapeDtypeStruct((B, N, KNB), jnp.int32),
        interpret=interpret,
    )(pos, posT)


def _pre_pallas(x2, pos2, Wv, bv, wka, interpret=False):
    M = x2.shape[0]
    TP = 512

    def body(x_ref, pos_ref, Wv_ref, bv_ref, wka_ref, v_ref, pk_ref):
        xt = x_ref[...]
        v_ref[...] = (jnp.dot(xt, Wv_ref[...], preferred_element_type=jnp.float32)
                      + bv_ref[...])
        ak = jnp.dot(xt, wka_ref[...], preferred_element_type=jnp.float32)
        pk_ref[...] = jnp.concatenate(
            [pos_ref[...], ak, jnp.zeros((TP, 124), jnp.float32)], axis=1)

    return pl.pallas_call(
        body,
        grid=(M // TP,),
        in_specs=[pl.BlockSpec((TP, DIM), lambda i: (i, 0)),
                  pl.BlockSpec((TP, 3), lambda i: (i, 0)),
                  pl.BlockSpec((DIM, DIM), lambda i: (0, 0)),
                  pl.BlockSpec((1, DIM), lambda i: (0, 0)),
                  pl.BlockSpec((DIM, 1), lambda i: (0, 0))],
        out_specs=[pl.BlockSpec((TP, DIM), lambda i: (i, 0)),
                   pl.BlockSpec((TP, 128), lambda i: (i, 0))],
        out_shape=[jax.ShapeDtypeStruct((M, DIM), jnp.float32),
                   jax.ShapeDtypeStruct((M, 128), jnp.float32)],
        interpret=interpret,
    )(x2, pos2, Wv, bv, wka)


def _gather_sc(vtab, pk, idxflat):
    MK = idxflat.shape[0]
    info = plsc.get_sparse_core_info()
    NC, NS = info.num_cores, info.num_subcores
    NW = NC * NS
    per_w = MK // NW
    CH = 128
    n_ch = per_w // CH
    mesh = plsc.VectorSubcoreMesh(core_axis_name="c", subcore_axis_name="s")

    @functools.partial(
        pl.kernel, mesh=mesh,
        out_type=[jax.ShapeDtypeStruct((MK, DIM), jnp.float32),
                  jax.ShapeDtypeStruct((MK, 128), jnp.float32)],
        scratch_types=[pltpu.VMEM((CH,), jnp.int32),
                       pltpu.VMEM((CH,), jnp.int32),
                       pltpu.VMEM((CH, DIM), jnp.float32),
                       pltpu.VMEM((CH, DIM), jnp.float32),
                       pltpu.VMEM((CH, 128), jnp.float32),
                       pltpu.VMEM((CH, 128), jnp.float32),
                       pltpu.SemaphoreType.DMA,
                       pltpu.SemaphoreType.DMA,
                       pltpu.SemaphoreType.DMA,
                       pltpu.SemaphoreType.DMA],
    )
    def k(vtab_hbm, pk_hbm, idx_hbm, vg_hbm, pkg_hbm,
          idx0, idx1, rows0, rows1, rows2a, rows2b,
          semg0, semg1, semw0, semw1):
        wid = lax.axis_index("s") * NC + lax.axis_index("c")
        base = wid * per_w
        bufs = [(idx0, rows0, rows2a, semg0, semw0),
                (idx1, rows1, rows2b, semg1, semw1)]
        wcps = [None, None]
        gcps = {}

        def issue(c):
            p = c & 1
            idxb, rb, r2b, semg, semw = bufs[p]
            if wcps[p] is not None:
                for wcp in wcps[p]:
                    wcp.wait()
                wcps[p] = None
            off = base + c * CH
            pltpu.sync_copy(idx_hbm.at[pl.ds(off, CH)], idxb)
            return (pltpu.async_copy(vtab_hbm.at[idxb], rb, semg),
                    pltpu.async_copy(pk_hbm.at[idxb], r2b, semg))

        gcps[0] = issue(0)
        for c in range(n_ch):
            p = c & 1
            if c + 1 < n_ch:
                gcps[c + 1] = issue(c + 1)
            for gcp in gcps.pop(c):
                gcp.wait()
            idxb, rb, r2b, semg, semw = bufs[p]
            off = base + c * CH
            wcps[p] = (pltpu.async_copy(rb, vg_hbm.at[pl.ds(off, CH)], semw),
                       pltpu.async_copy(r2b, pkg_hbm.at[pl.ds(off, CH)], semw))
        for p in (0, 1):
            if wcps[p] is not None:
                for wcp in wcps[p]:
                    wcp.wait()

    return k(vtab, pk, idxflat)


def _attn_ffn_pallas(vg, pkg, x2, pos2, Wp1, bp1, uT, Wcomb, bcomb,
                     g1, be1, g2, be2, Wf1, bf1, Wf2, bf2, interpret=False):
    M = x2.shape[0]
    TQ = 128
    TK = TQ * KNB

    def ln(r, g, b):
        mu = jnp.mean(r, axis=-1, keepdims=True)
        var = jnp.mean((r - mu) ** 2, axis=-1, keepdims=True)
        return (r - mu) / jnp.sqrt(var + 1e-5) * g + b

    def body(vg_ref, pkg_ref, x_ref, pos_ref, Wp1_ref, bp1_ref, uT_ref,
             Wcomb_ref, bcomb_ref, g1_ref, be1_ref, g2_ref, be2_ref,
             Wf1_ref, bf1_ref, Wf2_ref, bf2_ref, out_ref):
        vg3 = vg_ref[...].reshape(TQ, KNB, DIM)
        pkg3 = pkg_ref[...].reshape(TQ, KNB, 128)
        pos3 = pos_ref[...][:, None, :]
        pdx = pos3[:, :, 0] - pkg3[:, :, 0]
        pdy = pos3[:, :, 1] - pkg3[:, :, 1]
        pdz = pos3[:, :, 2] - pkg3[:, :, 2]
        w0 = Wp1_ref[0:1, :][None]
        w1 = Wp1_ref[1:2, :][None]
        w2 = Wp1_ref[2:3, :][None]
        h = (pdx[:, :, None] * w0 + pdy[:, :, None] * w1 + pdz[:, :, None] * w2
             + bp1_ref[...][None])
        h = jnp.maximum(h, 0.0)
        hu = jnp.sum(h * uT_ref[...][None], axis=-1)
        logits = hu - pkg3[:, :, 3]
        logits = logits - jnp.max(logits, axis=-1, keepdims=True)
        e = jnp.exp(logits)
        w = e / jnp.sum(e, axis=-1, keepdims=True)
        w3 = w[:, :, None]
        wv = jnp.sum(w3 * vg3, axis=1)
        s = jnp.sum(w3 * h, axis=1)
        cat = jnp.concatenate([wv, s], axis=-1)
        y = (jnp.dot(cat, Wcomb_ref[...], preferred_element_type=jnp.float32)
             + bcomb_ref[...])
        o1 = ln(y + x_ref[...], g1_ref[...], be1_ref[...])
        z = (jnp.dot(o1, Wf1_ref[...], preferred_element_type=jnp.float32)
             + bf1_ref[...])
        g = 0.5 * z * (1.0 + lax.erf(z * (2.0 ** -0.5)))
        f = (jnp.dot(g, Wf2_ref[...], preferred_element_type=jnp.float32)
             + bf2_ref[...])
        out_ref[...] = ln(o1 + f, g2_ref[...], be2_ref[...])

    const = lambda i: (0, 0)
    return pl.pallas_call(
        body,
        grid=(M // TQ,),
        in_specs=[pl.BlockSpec((TK, DIM), lambda i: (i, 0)),
                  pl.BlockSpec((TK, 128), lambda i: (i, 0)),
                  pl.BlockSpec((TQ, DIM), lambda i: (i, 0)),
                  pl.BlockSpec((TQ, 3), lambda i: (i, 0)),
                  pl.BlockSpec((3, DIM), const),
                  pl.BlockSpec((1, DIM), const),
                  pl.BlockSpec((1, DIM), const),
                  pl.BlockSpec((2 * DIM, DIM), const),
                  pl.BlockSpec((1, DIM), const),
                  pl.BlockSpec((1, DIM), const),
                  pl.BlockSpec((1, DIM), const),
                  pl.BlockSpec((1, DIM), const),
                  pl.BlockSpec((1, DIM), const),
                  pl.BlockSpec((DIM, 2 * DIM), const),
                  pl.BlockSpec((1, 2 * DIM), const),
                  pl.BlockSpec((2 * DIM, DIM), const),
                  pl.BlockSpec((1, DIM), const)],
        out_specs=pl.BlockSpec((TQ, DIM), lambda i: (i, 0)),
        out_shape=jax.ShapeDtypeStruct((M, DIM), jnp.float32),
        interpret=interpret,
    )(vg, pkg, x2, pos2, Wp1, bp1, uT, Wcomb, bcomb,
      g1, be1, g2, be2, Wf1, bf1, Wf2, bf2)


def kernel(x, pos, Wq, bq, Wk, bk, Wv, bv, Wp1, bp1, Wp2, bp2, Wa, ba, Wo, bo,
           g1, be1, g2, be2, Wf1, bf1, Wf2, bf2):
    B, N, C = x.shape
    M = B * N

    wka = Wk @ Wa
    uT = (Wp2 @ Wa).T
    Wcomb = jnp.concatenate([Wo, Wp2 @ Wo], axis=0)
    bcomb = (bp2 @ Wo + bo)[None]

    posT = jnp.transpose(pos, (0, 2, 1))
    idx = _knn_pallas(pos, posT)
    x2 = x.reshape(M, C)
    pos2 = pos.reshape(M, 3)
    vtab, pk = _pre_pallas(x2, pos2, Wv, bv[None], wka)
    idxflat = idx.reshape(M * KNB)
    vg, pkg = _gather_sc(vtab, pk, idxflat)
    out2 = _attn_ffn_pallas(
        vg, pkg, x2, pos2, Wp1, bp1[None], uT, Wcomb, bcomb,
        g1[None], be1[None], g2[None], be2[None],
        Wf1, bf1[None], Wf2, bf2[None])
    return out2.reshape(B, N, C)


# speedup vs baseline: 1.1214x; 1.1214x over previous
import functools

import jax
import jax.numpy as jnp
from jax import lax
from jax.experimental import pallas as pl
from jax.experimental.pallas import tpu as pltpu
from jax.experimental.pallas import tpu_sc as plsc

DIM = 256
KNB = 16


def _knn_pallas(pos, posT, interpret=False):
    B, N, _ = pos.shape
    TQ = 256

    def body(pos_ref, posT_ref, idx_ref):
        b = pl.program_id(0)
        pt = pos_ref[0]
        pT = posT_ref[0]
        dot = jnp.dot(pt, pT, preferred_element_type=jnp.float32)
        sq_r = jnp.sum(pT * pT, axis=0, keepdims=True)
        sq_t = jnp.sum(pt * pt, axis=1, keepdims=True)
        d2 = sq_t + sq_r - 2.0 * dot
        iota = lax.broadcasted_iota(jnp.int32, (TQ, N), 1).astype(jnp.float32)
        fn = jnp.float32(N)
        cols = []
        for _ in range(KNB):
            m = jnp.min(d2, axis=1, keepdims=True)
            cand = jnp.where(d2 == m, iota, fn)
            amin = jnp.min(cand, axis=1, keepdims=True)
            cols.append(amin)
            d2 = jnp.where(cand == amin, jnp.inf, d2)
        idxf = jnp.concatenate(cols, axis=1)
        idx_ref[0] = idxf.astype(jnp.int32) + b * N

    return pl.pallas_call(
        body,
        grid=(B, N // TQ),
        in_specs=[pl.BlockSpec((1, TQ, 3), lambda b, i: (b, i, 0)),
                  pl.BlockSpec((1, 3, N), lambda b, i: (b, 0, 0))],
        out_specs=pl.BlockSpec((1, TQ, KNB), lambda b, i: (b, i, 0)),
        out_shape=jax.ShapeDtypeStruct((B, N, KNB), jnp.int32),
        interpret=interpret,
    )(pos, posT)


def _pre_pallas(x2, pos2, Wv, bv, wka, interpret=False):
    M = x2.shape[0]
    TP = 512

    def body(x_ref, pos_ref, Wv_ref, bv_ref, wka_ref, v_ref, pk_ref):
        xt = x_ref[...]
        v_ref[...] = (jnp.dot(xt, Wv_ref[...], preferred_element_type=jnp.float32)
                      + bv_ref[...])
        ak = jnp.dot(xt, wka_ref[...], preferred_element_type=jnp.float32)
        pk_ref[...] = jnp.concatenate(
            [pos_ref[...], ak, jnp.zeros((TP, 124), jnp.float32)], axis=1)

    return pl.pallas_call(
        body,
        grid=(M // TP,),
        in_specs=[pl.BlockSpec((TP, DIM), lambda i: (i, 0)),
                  pl.BlockSpec((TP, 3), lambda i: (i, 0)),
                  pl.BlockSpec((DIM, DIM), lambda i: (0, 0)),
                  pl.BlockSpec((1, DIM), lambda i: (0, 0)),
                  pl.BlockSpec((DIM, 1), lambda i: (0, 0))],
        out_specs=[pl.BlockSpec((TP, DIM), lambda i: (i, 0)),
                   pl.BlockSpec((TP, 128), lambda i: (i, 0))],
        out_shape=[jax.ShapeDtypeStruct((M, DIM), jnp.float32),
                   jax.ShapeDtypeStruct((M, 128), jnp.float32)],
        interpret=interpret,
    )(x2, pos2, Wv, bv, wka)


def _gather_sc(vtab, pk, idxflat):
    MK = idxflat.shape[0]
    info = plsc.get_sparse_core_info()
    NC, NS = info.num_cores, info.num_subcores
    NW = NC * NS
    per_w = MK // NW
    CH = 128
    n_ch = per_w // CH
    mesh = plsc.VectorSubcoreMesh(core_axis_name="c", subcore_axis_name="s")

    @functools.partial(
        pl.kernel, mesh=mesh,
        out_type=[jax.ShapeDtypeStruct((MK, DIM), jnp.float32),
                  jax.ShapeDtypeStruct((MK, 128), jnp.float32)],
        scratch_types=[pltpu.VMEM((CH,), jnp.int32),
                       pltpu.VMEM((CH,), jnp.int32),
                       pltpu.VMEM((CH, DIM), jnp.float32),
                       pltpu.VMEM((CH, DIM), jnp.float32),
                       pltpu.VMEM((CH, 128), jnp.float32),
                       pltpu.VMEM((CH, 128), jnp.float32),
                       pltpu.SemaphoreType.DMA,
                       pltpu.SemaphoreType.DMA,
                       pltpu.SemaphoreType.DMA,
                       pltpu.SemaphoreType.DMA],
    )
    def k(vtab_hbm, pk_hbm, idx_hbm, vg_hbm, pkg_hbm,
          idx0, idx1, rows0, rows1, rows2a, rows2b,
          semg0, semg1, semw0, semw1):
        wid = lax.axis_index("s") * NC + lax.axis_index("c")
        base = wid * per_w
        bufs = [(idx0, rows0, rows2a, semg0, semw0),
                (idx1, rows1, rows2b, semg1, semw1)]
        wcps = [None, None]
        gcps = {}

        def issue(c):
            p = c & 1
            idxb, rb, r2b, semg, semw = bufs[p]
            if wcps[p] is not None:
                for wcp in wcps[p]:
                    wcp.wait()
                wcps[p] = None
            off = base + c * CH
            pltpu.sync_copy(idx_hbm.at[pl.ds(off, CH)], idxb)
            return (pltpu.async_copy(vtab_hbm.at[idxb], rb, semg),
                    pltpu.async_copy(pk_hbm.at[idxb], r2b, semg))

        gcps[0] = issue(0)
        for c in range(n_ch):
            p = c & 1
            if c + 1 < n_ch:
                gcps[c + 1] = issue(c + 1)
            for gcp in gcps.pop(c):
                gcp.wait()
            idxb, rb, r2b, semg, semw = bufs[p]
            off = base + c * CH
            wcps[p] = (pltpu.async_copy(rb, vg_hbm.at[pl.ds(off, CH)], semw),
                       pltpu.async_copy(r2b, pkg_hbm.at[pl.ds(off, CH)], semw))
        for p in (0, 1):
            if wcps[p] is not None:
                for wcp in wcps[p]:
                    wcp.wait()

    return k(vtab, pk, idxflat)


def _attn_ffn_pallas(vg, pkg, x2, pos2, Wp1, bp1, uT, Wcomb, bcomb,
                     g1, be1, g2, be2, Wf1, bf1, Wf2, bf2, interpret=False):
    M = x2.shape[0]
    TQ = 128
    TK = TQ * KNB

    def ln(r, g, b):
        mu = jnp.mean(r, axis=-1, keepdims=True)
        var = jnp.mean((r - mu) ** 2, axis=-1, keepdims=True)
        return (r - mu) / jnp.sqrt(var + 1e-5) * g + b

    def body(vg_ref, pkg_ref, x_ref, pos_ref, Wp1_ref, bp1_ref, uT_ref,
             Wcomb_ref, bcomb_ref, g1_ref, be1_ref, g2_ref, be2_ref,
             Wf1_ref, bf1_ref, Wf2_ref, bf2_ref, out_ref):
        vg3 = vg_ref[...].reshape(TQ, KNB, DIM)
        pkg3 = pkg_ref[...].reshape(TQ, KNB, 128)
        pos3 = pos_ref[...][:, None, :]
        pd = pos3 - pkg3[:, :, 0:3]
        w0 = Wp1_ref[0:1, :][None]
        w1 = Wp1_ref[1:2, :][None]
        w2 = Wp1_ref[2:3, :][None]
        h = (pd[:, :, 0:1] * w0 + pd[:, :, 1:2] * w1 + pd[:, :, 2:3] * w2
             + bp1_ref[...][None])
        h = jnp.maximum(h, 0.0)
        hu = jnp.sum(h * uT_ref[...][None], axis=-1)
        logits = hu - pkg3[:, :, 3]
        logits = logits - jnp.max(logits, axis=-1, keepdims=True)
        e = jnp.exp(logits)
        w = e / jnp.sum(e, axis=-1, keepdims=True)
        w3 = w[:, :, None]
        wv = jnp.sum(w3 * vg3, axis=1)
        s = jnp.sum(w3 * h, axis=1)
        cat = jnp.concatenate([wv, s], axis=-1)
        y = (jnp.dot(cat, Wcomb_ref[...], preferred_element_type=jnp.float32)
             + bcomb_ref[...])
        o1 = ln(y + x_ref[...], g1_ref[...], be1_ref[...])
        z = (jnp.dot(o1, Wf1_ref[...], preferred_element_type=jnp.float32)
             + bf1_ref[...])
        g = 0.5 * z * (1.0 + lax.erf(z * (2.0 ** -0.5)))
        f = (jnp.dot(g, Wf2_ref[...], preferred_element_type=jnp.float32)
             + bf2_ref[...])
        out_ref[...] = ln(o1 + f, g2_ref[...], be2_ref[...])

    const = lambda i: (0, 0)
    return pl.pallas_call(
        body,
        grid=(M // TQ,),
        in_specs=[pl.BlockSpec((TK, DIM), lambda i: (i, 0)),
                  pl.BlockSpec((TK, 128), lambda i: (i, 0)),
                  pl.BlockSpec((TQ, DIM), lambda i: (i, 0)),
                  pl.BlockSpec((TQ, 3), lambda i: (i, 0)),
                  pl.BlockSpec((3, DIM), const),
                  pl.BlockSpec((1, DIM), const),
                  pl.BlockSpec((1, DIM), const),
                  pl.BlockSpec((2 * DIM, DIM), const),
                  pl.BlockSpec((1, DIM), const),
                  pl.BlockSpec((1, DIM), const),
                  pl.BlockSpec((1, DIM), const),
                  pl.BlockSpec((1, DIM), const),
                  pl.BlockSpec((1, DIM), const),
                  pl.BlockSpec((DIM, 2 * DIM), const),
                  pl.BlockSpec((1, 2 * DIM), const),
                  pl.BlockSpec((2 * DIM, DIM), const),
                  pl.BlockSpec((1, DIM), const)],
        out_specs=pl.BlockSpec((TQ, DIM), lambda i: (i, 0)),
        out_shape=jax.ShapeDtypeStruct((M, DIM), jnp.float32),
        interpret=interpret,
    )(vg, pkg, x2, pos2, Wp1, bp1, uT, Wcomb, bcomb,
      g1, be1, g2, be2, Wf1, bf1, Wf2, bf2)


def kernel(x, pos, Wq, bq, Wk, bk, Wv, bv, Wp1, bp1, Wp2, bp2, Wa, ba, Wo, bo,
           g1, be1, g2, be2, Wf1, bf1, Wf2, bf2):
    B, N, C = x.shape
    M = B * N

    wka = Wk @ Wa
    uT = (Wp2 @ Wa).T
    Wcomb = jnp.concatenate([Wo, Wp2 @ Wo], axis=0)
    bcomb = (bp2 @ Wo + bo)[None]

    posT = jnp.transpose(pos, (0, 2, 1))
    idx = _knn_pallas(pos, posT)
    x2 = x.reshape(M, C)
    pos2 = pos.reshape(M, 3)
    vtab, pk = _pre_pallas(x2, pos2, Wv, bv[None], wka)
    idxflat = idx.reshape(M * KNB)
    vg, pkg = _gather_sc(vtab, pk, idxflat)
    out2 = _attn_ffn_pallas(
        vg, pkg, x2, pos2, Wp1, bp1[None], uT, Wcomb, bcomb,
        g1[None], be1[None], g2[None], be2[None],
        Wf1, bf1[None], Wf2, bf2[None])
    return out2.reshape(B, N, C)


# speedup vs baseline: 1.1725x; 1.0455x over previous
import functools

import jax
import jax.numpy as jnp
from jax import lax
from jax.experimental import pallas as pl
from jax.experimental.pallas import tpu as pltpu
from jax.experimental.pallas import tpu_sc as plsc

DIM = 256
KNB = 16


def _knn_pallas(pos, posT, interpret=False):
    B, N, _ = pos.shape
    TQ = 256

    def body(pos_ref, posT_ref, idx_ref):
        b = pl.program_id(0)
        pt = pos_ref[0]
        pT = posT_ref[0]
        dot = jnp.dot(pt, pT, preferred_element_type=jnp.float32)
        sq_r = jnp.sum(pT * pT, axis=0, keepdims=True)
        sq_t = jnp.sum(pt * pt, axis=1, keepdims=True)
        d2 = sq_t + sq_r - 2.0 * dot
        iota = lax.broadcasted_iota(jnp.int32, (TQ, N), 1).astype(jnp.float32)
        fn = jnp.float32(N)
        cols = []
        for _ in range(KNB):
            m = jnp.min(d2, axis=1, keepdims=True)
            cand = jnp.where(d2 == m, iota, fn)
            amin = jnp.min(cand, axis=1, keepdims=True)
            cols.append(amin)
            d2 = jnp.where(cand == amin, jnp.inf, d2)
        idxf = jnp.concatenate(cols, axis=1)
        idx_ref[0] = idxf.astype(jnp.int32) + b * N

    return pl.pallas_call(
        body,
        grid=(B, N // TQ),
        in_specs=[pl.BlockSpec((1, TQ, 3), lambda b, i: (b, i, 0)),
                  pl.BlockSpec((1, 3, N), lambda b, i: (b, 0, 0))],
        out_specs=pl.BlockSpec((1, TQ, KNB), lambda b, i: (b, i, 0)),
        out_shape=jax.ShapeDtypeStruct((B, N, KNB), jnp.int32),
        interpret=interpret,
    )(pos, posT)


def _pre_pallas(x2, pos2, Wv, bv, wka, interpret=False):
    M = x2.shape[0]
    TP = 512

    def body(x_ref, pos_ref, Wv_ref, bv_ref, wka_ref, v_ref, pk_ref):
        xt = x_ref[...]
        v_ref[...] = (jnp.dot(xt, Wv_ref[...], preferred_element_type=jnp.float32)
                      + bv_ref[...])
        ak = jnp.dot(xt, wka_ref[...], preferred_element_type=jnp.float32)
        pk_ref[...] = jnp.concatenate(
            [pos_ref[...], ak, jnp.zeros((TP, 124), jnp.float32)], axis=1)

    return pl.pallas_call(
        body,
        grid=(M // TP,),
        in_specs=[pl.BlockSpec((TP, DIM), lambda i: (i, 0)),
                  pl.BlockSpec((TP, 3), lambda i: (i, 0)),
                  pl.BlockSpec((DIM, DIM), lambda i: (0, 0)),
                  pl.BlockSpec((1, DIM), lambda i: (0, 0)),
                  pl.BlockSpec((DIM, 1), lambda i: (0, 0))],
        out_specs=[pl.BlockSpec((TP, DIM), lambda i: (i, 0)),
                   pl.BlockSpec((TP, 128), lambda i: (i, 0))],
        out_shape=[jax.ShapeDtypeStruct((M, DIM), jnp.float32),
                   jax.ShapeDtypeStruct((M, 128), jnp.float32)],
        interpret=interpret,
    )(x2, pos2, Wv, bv, wka)


def _gather_sc(vtab, pk, idxflat):
    MK = idxflat.shape[0]
    info = plsc.get_sparse_core_info()
    NC, NS = info.num_cores, info.num_subcores
    NW = NC * NS
    per_w = MK // NW
    CH = 128
    n_ch = per_w // CH
    mesh = plsc.VectorSubcoreMesh(core_axis_name="c", subcore_axis_name="s")

    @functools.partial(
        pl.kernel, mesh=mesh,
        out_type=[jax.ShapeDtypeStruct((MK, DIM), jnp.float32),
                  jax.ShapeDtypeStruct((MK, 128), jnp.float32)],
        scratch_types=[pltpu.VMEM((CH,), jnp.int32),
                       pltpu.VMEM((CH,), jnp.int32),
                       pltpu.VMEM((CH, DIM), jnp.float32),
                       pltpu.VMEM((CH, DIM), jnp.float32),
                       pltpu.VMEM((CH, 128), jnp.float32),
                       pltpu.VMEM((CH, 128), jnp.float32),
                       pltpu.SemaphoreType.DMA,
                       pltpu.SemaphoreType.DMA,
                       pltpu.SemaphoreType.DMA,
                       pltpu.SemaphoreType.DMA],
    )
    def k(vtab_hbm, pk_hbm, idx_hbm, vg_hbm, pkg_hbm,
          idx0, idx1, rows0, rows1, rows2a, rows2b,
          semg0, semg1, semw0, semw1):
        wid = lax.axis_index("s") * NC + lax.axis_index("c")
        base = wid * per_w
        bufs = [(idx0, rows0, rows2a, semg0, semw0),
                (idx1, rows1, rows2b, semg1, semw1)]
        wcps = [None, None]
        gcps = {}

        def issue(c):
            p = c & 1
            idxb, rb, r2b, semg, semw = bufs[p]
            if wcps[p] is not None:
                for wcp in wcps[p]:
                    wcp.wait()
                wcps[p] = None
            off = base + c * CH
            pltpu.sync_copy(idx_hbm.at[pl.ds(off, CH)], idxb)
            return (pltpu.async_copy(vtab_hbm.at[idxb], rb, semg),
                    pltpu.async_copy(pk_hbm.at[idxb], r2b, semg))

        gcps[0] = issue(0)
        for c in range(n_ch):
            p = c & 1
            if c + 1 < n_ch:
                gcps[c + 1] = issue(c + 1)
            for gcp in gcps.pop(c):
                gcp.wait()
            idxb, rb, r2b, semg, semw = bufs[p]
            off = base + c * CH
            wcps[p] = (pltpu.async_copy(rb, vg_hbm.at[pl.ds(off, CH)], semw),
                       pltpu.async_copy(r2b, pkg_hbm.at[pl.ds(off, CH)], semw))
        for p in (0, 1):
            if wcps[p] is not None:
                for wcp in wcps[p]:
                    wcp.wait()

    return k(vtab, pk, idxflat)


def _attn_ffn_pallas(vg, pkg, x2, pos2, Wp1, bp1, uT, Wcomb, bcomb,
                     g1, be1, g2, be2, Wf1, bf1, Wf2, bf2, interpret=False):
    M = x2.shape[0]
    TQ = 128
    TK = TQ * KNB

    def ln(r, g, b):
        mu = jnp.mean(r, axis=-1, keepdims=True)
        var = jnp.mean((r - mu) ** 2, axis=-1, keepdims=True)
        return (r - mu) / jnp.sqrt(var + 1e-5) * g + b

    def body(vg_ref, pkg_ref, x_ref, pos_ref, Wp1_ref, bp1_ref, uT_ref,
             Wcomb_ref, bcomb_ref, g1_ref, be1_ref, g2_ref, be2_ref,
             Wf1_ref, bf1_ref, Wf2_ref, bf2_ref, out_ref):
        vg3 = vg_ref[...].reshape(TQ, KNB, DIM)
        pkg3 = pkg_ref[...].reshape(TQ, KNB, 128)
        pos3 = pos_ref[...][:, None, :]
        pd = pos3 - pkg3[:, :, 0:3]
        w0 = Wp1_ref[0:1, :][None]
        w1 = Wp1_ref[1:2, :][None]
        w2 = Wp1_ref[2:3, :][None]
        h = (pd[:, :, 0:1] * w0 + pd[:, :, 1:2] * w1 + pd[:, :, 2:3] * w2
             + bp1_ref[...][None])
        h = jnp.maximum(h, 0.0)
        hu = jnp.sum(h * uT_ref[...][None], axis=-1)
        logits = hu - pkg3[:, :, 3]
        logits = logits - jnp.max(logits, axis=-1, keepdims=True)
        e = jnp.exp(logits)
        w = e / jnp.sum(e, axis=-1, keepdims=True)
        w3 = w[:, :, None]
        wv = jnp.sum(w3 * vg3, axis=1)
        s = jnp.sum(w3 * h, axis=1)
        cat = jnp.concatenate([wv, s], axis=-1)
        y = (jnp.dot(cat, Wcomb_ref[...], preferred_element_type=jnp.float32)
             + bcomb_ref[...])
        o1 = ln(y + x_ref[...], g1_ref[...], be1_ref[...])
        z = (jnp.dot(o1, Wf1_ref[...], preferred_element_type=jnp.float32)
             + bf1_ref[...])
        g = 0.5 * z * (1.0 + lax.erf(z * (2.0 ** -0.5)))
        f = (jnp.dot(g, Wf2_ref[...], preferred_element_type=jnp.float32)
             + bf2_ref[...])
        out_ref[...] = ln(o1 + f, g2_ref[...], be2_ref[...])

    const = lambda i: (0, 0)
    return pl.pallas_call(
        body,
        grid=(M // TQ,),
        in_specs=[pl.BlockSpec((TK, DIM), lambda i: (i, 0)),
                  pl.BlockSpec((TK, 128), lambda i: (i, 0)),
                  pl.BlockSpec((TQ, DIM), lambda i: (i, 0)),
                  pl.BlockSpec((TQ, 3), lambda i: (i, 0)),
                  pl.BlockSpec((3, DIM), const),
                  pl.BlockSpec((1, DIM), const),
                  pl.BlockSpec((1, DIM), const),
                  pl.BlockSpec((2 * DIM, DIM), const),
                  pl.BlockSpec((1, DIM), const),
                  pl.BlockSpec((1, DIM), const),
                  pl.BlockSpec((1, DIM), const),
                  pl.BlockSpec((1, DIM), const),
                  pl.BlockSpec((1, DIM), const),
                  pl.BlockSpec((DIM, 2 * DIM), const),
                  pl.BlockSpec((1, 2 * DIM), const),
                  pl.BlockSpec((2 * DIM, DIM), const),
                  pl.BlockSpec((1, DIM), const)],
        out_specs=pl.BlockSpec((TQ, DIM), lambda i: (i, 0)),
        out_shape=jax.ShapeDtypeStruct((M, DIM), jnp.float32),
        interpret=interpret,
    )(vg, pkg, x2, pos2, Wp1, bp1, uT, Wcomb, bcomb,
      g1, be1, g2, be2, Wf1, bf1, Wf2, bf2)


def kernel(x, pos, Wq, bq, Wk, bk, Wv, bv, Wp1, bp1, Wp2, bp2, Wa, ba, Wo, bo,
           g1, be1, g2, be2, Wf1, bf1, Wf2, bf2):
    B, N, C = x.shape
    M = B * N

    wka = Wk @ Wa
    uT = (Wp2 @ Wa).T
    Wcomb = jnp.concatenate([Wo, Wp2 @ Wo], axis=0)
    bcomb = (bp2 @ Wo + bo)[None]

    posT = jnp.transpose(pos, (0, 2, 1))
    idx = _knn_pallas(pos, posT)
    x2 = x.reshape(M, C)
    pos2 = pos.reshape(M, 3)
    vtab, pk = _pre_pallas(x2, pos2, Wv, bv[None], wka)
    idxflat = idx.reshape(M * KNB)

    Mh = M // 2
    outs = []
    for s in (0, 1):
        idx_h = lax.dynamic_slice_in_dim(idxflat, s * Mh * KNB, Mh * KNB)
        vg, pkg = _gather_sc(vtab, pk, idx_h)
        out_h = _attn_ffn_pallas(
            vg, pkg,
            lax.dynamic_slice_in_dim(x2, s * Mh, Mh),
            lax.dynamic_slice_in_dim(pos2, s * Mh, Mh),
            Wp1, bp1[None], uT, Wcomb, bcomb,
            g1[None], be1[None], g2[None], be2[None],
            Wf1, bf1[None], Wf2, bf2[None])
        outs.append(out_h)
    out2 = jnp.concatenate(outs, axis=0)
    return out2.reshape(B, N, C)


# speedup vs baseline: 1.2004x; 1.0238x over previous
import functools

import jax
import jax.numpy as jnp
from jax import lax
from jax.experimental import pallas as pl
from jax.experimental.pallas import tpu as pltpu
from jax.experimental.pallas import tpu_sc as plsc

DIM = 256
KNB = 16


def _knn_pallas(pos, posT, interpret=False):
    B, N, _ = pos.shape
    TQ = 256

    def body(pos_ref, posT_ref, idx_ref):
        b = pl.program_id(0)
        pt = pos_ref[0]
        pT = posT_ref[0]
        dot = jnp.dot(pt, pT, preferred_element_type=jnp.float32)
        sq_r = jnp.sum(pT * pT, axis=0, keepdims=True)
        sq_t = jnp.sum(pt * pt, axis=1, keepdims=True)
        d2 = sq_t + sq_r - 2.0 * dot
        iota = lax.broadcasted_iota(jnp.int32, (TQ, N), 1).astype(jnp.float32)
        fn = jnp.float32(N)
        cols = []
        for _ in range(KNB):
            m = jnp.min(d2, axis=1, keepdims=True)
            cand = jnp.where(d2 == m, iota, fn)
            amin = jnp.min(cand, axis=1, keepdims=True)
            cols.append(amin)
            d2 = jnp.where(cand == amin, jnp.inf, d2)
        idxf = jnp.concatenate(cols, axis=1)
        idx_ref[0] = idxf.astype(jnp.int32) + b * N

    return pl.pallas_call(
        body,
        grid=(B, N // TQ),
        in_specs=[pl.BlockSpec((1, TQ, 3), lambda b, i: (b, i, 0)),
                  pl.BlockSpec((1, 3, N), lambda b, i: (b, 0, 0))],
        out_specs=pl.BlockSpec((1, TQ, KNB), lambda b, i: (b, i, 0)),
        out_shape=jax.ShapeDtypeStruct((B, N, KNB), jnp.int32),
        interpret=interpret,
    )(pos, posT)


def _pre_pallas(x2, pos2, Wv, bv, wka, interpret=False):
    M = x2.shape[0]
    TP = 512

    def body(x_ref, pos_ref, Wv_ref, bv_ref, wka_ref, v_ref, pk_ref):
        xt = x_ref[...]
        v_ref[...] = (jnp.dot(xt, Wv_ref[...], preferred_element_type=jnp.float32)
                      + bv_ref[...])
        ak = jnp.dot(xt, wka_ref[...], preferred_element_type=jnp.float32)
        pk_ref[...] = jnp.concatenate(
            [pos_ref[...], ak, jnp.zeros((TP, 124), jnp.float32)], axis=1)

    return pl.pallas_call(
        body,
        grid=(M // TP,),
        in_specs=[pl.BlockSpec((TP, DIM), lambda i: (i, 0)),
                  pl.BlockSpec((TP, 3), lambda i: (i, 0)),
                  pl.BlockSpec((DIM, DIM), lambda i: (0, 0)),
                  pl.BlockSpec((1, DIM), lambda i: (0, 0)),
                  pl.BlockSpec((DIM, 1), lambda i: (0, 0))],
        out_specs=[pl.BlockSpec((TP, DIM), lambda i: (i, 0)),
                   pl.BlockSpec((TP, 128), lambda i: (i, 0))],
        out_shape=[jax.ShapeDtypeStruct((M, DIM), jnp.float32),
                   jax.ShapeDtypeStruct((M, 128), jnp.float32)],
        interpret=interpret,
    )(x2, pos2, Wv, bv, wka)


def _gather_sc(vtab, pk, idxflat):
    MK = idxflat.shape[0]
    info = plsc.get_sparse_core_info()
    NC, NS = info.num_cores, info.num_subcores
    NW = NC * NS
    per_w = MK // NW
    CH = 128
    n_ch = per_w // CH
    mesh = plsc.VectorSubcoreMesh(core_axis_name="c", subcore_axis_name="s")

    @functools.partial(
        pl.kernel, mesh=mesh,
        out_type=[jax.ShapeDtypeStruct((MK, DIM), jnp.float32),
                  jax.ShapeDtypeStruct((MK, 128), jnp.float32)],
        scratch_types=[pltpu.VMEM((CH,), jnp.int32),
                       pltpu.VMEM((CH,), jnp.int32),
                       pltpu.VMEM((CH, DIM), jnp.float32),
                       pltpu.VMEM((CH, DIM), jnp.float32),
                       pltpu.VMEM((CH, 128), jnp.float32),
                       pltpu.VMEM((CH, 128), jnp.float32),
                       pltpu.SemaphoreType.DMA,
                       pltpu.SemaphoreType.DMA,
                       pltpu.SemaphoreType.DMA,
                       pltpu.SemaphoreType.DMA],
    )
    def k(vtab_hbm, pk_hbm, idx_hbm, vg_hbm, pkg_hbm,
          idx0, idx1, rows0, rows1, rows2a, rows2b,
          semg0, semg1, semw0, semw1):
        wid = lax.axis_index("s") * NC + lax.axis_index("c")
        base = wid * per_w
        bufs = [(idx0, rows0, rows2a, semg0, semw0),
                (idx1, rows1, rows2b, semg1, semw1)]
        wcps = [None, None]
        gcps = {}

        def issue(c):
            p = c & 1
            idxb, rb, r2b, semg, semw = bufs[p]
            if wcps[p] is not None:
                for wcp in wcps[p]:
                    wcp.wait()
                wcps[p] = None
            off = base + c * CH
            pltpu.sync_copy(idx_hbm.at[pl.ds(off, CH)], idxb)
            return (pltpu.async_copy(vtab_hbm.at[idxb], rb, semg),
                    pltpu.async_copy(pk_hbm.at[idxb], r2b, semg))

        gcps[0] = issue(0)
        for c in range(n_ch):
            p = c & 1
            if c + 1 < n_ch:
                gcps[c + 1] = issue(c + 1)
            for gcp in gcps.pop(c):
                gcp.wait()
            idxb, rb, r2b, semg, semw = bufs[p]
            off = base + c * CH
            wcps[p] = (pltpu.async_copy(rb, vg_hbm.at[pl.ds(off, CH)], semw),
                       pltpu.async_copy(r2b, pkg_hbm.at[pl.ds(off, CH)], semw))
        for p in (0, 1):
            if wcps[p] is not None:
                for wcp in wcps[p]:
                    wcp.wait()

    return k(vtab, pk, idxflat)


def _attn_ffn_pallas(vg, pkg, x2, pos2, Wp1, bp1, uT, Wcomb, bcomb,
                     g1, be1, g2, be2, Wf1, bf1, Wf2, bf2, interpret=False):
    M = x2.shape[0]
    TQ = 128
    TK = TQ * KNB

    def ln(r, g, b):
        mu = jnp.mean(r, axis=-1, keepdims=True)
        var = jnp.mean((r - mu) ** 2, axis=-1, keepdims=True)
        return (r - mu) / jnp.sqrt(var + 1e-5) * g + b

    def body(vg_ref, pkg_ref, x_ref, pos_ref, Wp1_ref, bp1_ref, uT_ref,
             Wcomb_ref, bcomb_ref, g1_ref, be1_ref, g2_ref, be2_ref,
             Wf1_ref, bf1_ref, Wf2_ref, bf2_ref, out_ref):
        vg3 = vg_ref[...].reshape(TQ, KNB, DIM)
        pkg3 = pkg_ref[...].reshape(TQ, KNB, 128)
        pos3 = pos_ref[...][:, None, :]
        pd = pos3 - pkg3[:, :, 0:3]
        w0 = Wp1_ref[0:1, :][None]
        w1 = Wp1_ref[1:2, :][None]
        w2 = Wp1_ref[2:3, :][None]
        h = (pd[:, :, 0:1] * w0 + pd[:, :, 1:2] * w1 + pd[:, :, 2:3] * w2
             + bp1_ref[...][None])
        h = jnp.maximum(h, 0.0)
        hu = jnp.sum(h * uT_ref[...][None], axis=-1)
        logits = hu - pkg3[:, :, 3]
        logits = logits - jnp.max(logits, axis=-1, keepdims=True)
        e = jnp.exp(logits)
        w = e / jnp.sum(e, axis=-1, keepdims=True)
        w3 = w[:, :, None]
        wv = jnp.sum(w3 * vg3, axis=1)
        s = jnp.sum(w3 * h, axis=1)
        cat = jnp.concatenate([wv, s], axis=-1)
        y = (jnp.dot(cat, Wcomb_ref[...], preferred_element_type=jnp.float32)
             + bcomb_ref[...])
        o1 = ln(y + x_ref[...], g1_ref[...], be1_ref[...])
        z = (jnp.dot(o1, Wf1_ref[...], preferred_element_type=jnp.float32)
             + bf1_ref[...])
        g = 0.5 * z * (1.0 + lax.erf(z * (2.0 ** -0.5)))
        f = (jnp.dot(g, Wf2_ref[...], preferred_element_type=jnp.float32)
             + bf2_ref[...])
        out_ref[...] = ln(o1 + f, g2_ref[...], be2_ref[...])

    const = lambda i: (0, 0)
    return pl.pallas_call(
        body,
        grid=(M // TQ,),
        in_specs=[pl.BlockSpec((TK, DIM), lambda i: (i, 0)),
                  pl.BlockSpec((TK, 128), lambda i: (i, 0)),
                  pl.BlockSpec((TQ, DIM), lambda i: (i, 0)),
                  pl.BlockSpec((TQ, 3), lambda i: (i, 0)),
                  pl.BlockSpec((3, DIM), const),
                  pl.BlockSpec((1, DIM), const),
                  pl.BlockSpec((1, DIM), const),
                  pl.BlockSpec((2 * DIM, DIM), const),
                  pl.BlockSpec((1, DIM), const),
                  pl.BlockSpec((1, DIM), const),
                  pl.BlockSpec((1, DIM), const),
                  pl.BlockSpec((1, DIM), const),
                  pl.BlockSpec((1, DIM), const),
                  pl.BlockSpec((DIM, 2 * DIM), const),
                  pl.BlockSpec((1, 2 * DIM), const),
                  pl.BlockSpec((2 * DIM, DIM), const),
                  pl.BlockSpec((1, DIM), const)],
        out_specs=pl.BlockSpec((TQ, DIM), lambda i: (i, 0)),
        out_shape=jax.ShapeDtypeStruct((M, DIM), jnp.float32),
        interpret=interpret,
    )(vg, pkg, x2, pos2, Wp1, bp1, uT, Wcomb, bcomb,
      g1, be1, g2, be2, Wf1, bf1, Wf2, bf2)


def kernel(x, pos, Wq, bq, Wk, bk, Wv, bv, Wp1, bp1, Wp2, bp2, Wa, ba, Wo, bo,
           g1, be1, g2, be2, Wf1, bf1, Wf2, bf2):
    B, N, C = x.shape
    M = B * N

    wka = Wk @ Wa
    uT = (Wp2 @ Wa).T
    Wcomb = jnp.concatenate([Wo, Wp2 @ Wo], axis=0)
    bcomb = (bp2 @ Wo + bo)[None]

    posT = jnp.transpose(pos, (0, 2, 1))
    idx = _knn_pallas(pos, posT)
    x2 = x.reshape(M, C)
    pos2 = pos.reshape(M, 3)
    vtab, pk = _pre_pallas(x2, pos2, Wv, bv[None], wka)
    idxflat = idx.reshape(M * KNB)

    Mh = M // 4
    outs = []
    for s in (0, 1, 2, 3):
        idx_h = lax.dynamic_slice_in_dim(idxflat, s * Mh * KNB, Mh * KNB)
        vg, pkg = _gather_sc(vtab, pk, idx_h)
        out_h = _attn_ffn_pallas(
            vg, pkg,
            lax.dynamic_slice_in_dim(x2, s * Mh, Mh),
            lax.dynamic_slice_in_dim(pos2, s * Mh, Mh),
            Wp1, bp1[None], uT, Wcomb, bcomb,
            g1[None], be1[None], g2[None], be2[None],
            Wf1, bf1[None], Wf2, bf2[None])
        outs.append(out_h)
    out2 = jnp.concatenate(outs, axis=0)
    return out2.reshape(B, N, C)
